# baseline (speedup 1.0000x reference)
import jax
import jax.numpy as jnp
from jax import lax
from jax.experimental import pallas as pl
from jax.experimental.pallas import tpu as pltpu, tpu_sc as plsc

VOCAB = 1000000
B = 16384
L = 50
E = 64
PAD = 64
NC = 2
NS = 16
NW = NC * NS
BPW = B // NW
CB = 4
NCH = BPW // CB
ROT = 4
LANES = 16
NG = PAD // LANES
EUN = 4


def _sc_body(center_hbm, ctx_hbm, vflat_hbm, u_hbm, out_hbm,
             cidx_v, ctx_v, ubs, vbs, vis, obs, sems, osems):
    cid = lax.axis_index("c")
    sid = lax.axis_index("s")
    wid = sid * NC + cid
    base = wid * BPW

    pltpu.sync_copy(center_hbm.at[pl.ds(base, BPW)], cidx_v)
    pltpu.sync_copy(ctx_hbm.at[pl.ds(base, BPW)], ctx_v)

    iota = lax.iota(jnp.int32, LANES)
    rowg = [iota + g * LANES for g in range(NG)]
    evec = [(iota + g * LANES) * VOCAB for g in range(NG)]

    def issue(c, ub, vb, vi, sem):
        for i in range(CB):
            r = plsc.load_gather(cidx_v, [jnp.full((LANES,), c * CB + i,
                                                   jnp.int32)])
            for g in range(NG):
                vi[i, pl.ds(g * LANES, LANES)] = evec[g] + r
        hs = []
        for i in range(CB):
            hs.append(pltpu.async_copy(u_hbm.at[ctx_v.at[c * CB + i]],
                                       ub.at[pl.ds(i * PAD, L)], sem))
            hs.append(pltpu.async_copy(vflat_hbm.at[vi.at[i]], vb.at[i],
                                       sem))
        return hs

    def compute(c, ub, vb, ob, osem):
        for i in range(CB):
            rows = [rowg[g] + i * PAD for g in range(NG)]
            ifull = jnp.full((LANES,), i, jnp.int32)

            def ebody(t4, accs, rows=rows, ifull=ifull):
                for r in range(EUN):
                    col = (iota + (t4 * EUN + r)) & (E - 1)
                    s = plsc.load_gather(vb, [ifull, col])
                    accs = tuple(
                        accs[g] + plsc.load_gather(ub, [rows[g], col]) * s
                        for g in range(NG))
                return accs

            accs = lax.fori_loop(
                0, E // EUN, ebody,
                tuple(jnp.zeros((LANES,), jnp.float32) for _ in range(NG)))
            for g in range(NG):
                ob[i, pl.ds(g * LANES, LANES)] = accs[g]

        return pltpu.async_copy(ob, out_hbm.at[pl.ds(base + c * CB, CB)],
                                osem)

    def outer(cr, carry):
        c0 = cr * ROT
        hs = [issue(c0 + q, ubs[q], vbs[q], vis[q], sems[q])
              for q in range(ROT)]
        ohs = []
        for q in range(ROT):
            for h in hs[q]:
                h.wait()
            ohs.append(compute(c0 + q, ubs[q], vbs[q], obs[q], osems[q]))
        for oh in ohs:
            oh.wait()
        return carry

    lax.fori_loop(0, NCH // ROT, outer, 0)


def kernel(center, context_negative, v_table, u_table):
    vflat = v_table.T.reshape(VOCAB * E)
    mesh = plsc.VectorSubcoreMesh(core_axis_name="c", subcore_axis_name="s")
    padded = pl.kernel(
        _sc_body,
        out_type=jax.ShapeDtypeStruct((B, PAD), jnp.float32),
        mesh=mesh,
        compiler_params=pltpu.CompilerParams(needs_layout_passes=False,
                                             use_tc_tiling_on_sc=False),
        scratch_types=[
            pltpu.VMEM((BPW,), jnp.int32),
            pltpu.VMEM((BPW, L), jnp.int32),
            [pltpu.VMEM((CB * PAD, E), jnp.float32) for _ in range(ROT)],
            [pltpu.VMEM((CB, E), jnp.float32) for _ in range(ROT)],
            [pltpu.VMEM((CB, E), jnp.int32) for _ in range(ROT)],
            [pltpu.VMEM((CB, PAD), jnp.float32) for _ in range(ROT)],
            [pltpu.SemaphoreType.DMA for _ in range(ROT)],
            [pltpu.SemaphoreType.DMA for _ in range(ROT)],
        ],
    )(center.reshape(B), context_negative, vflat, u_table)
    return padded[:, :L].reshape(B, 1, L)


# speedup vs baseline: 4.3377x; 4.3377x over previous
import jax
import jax.numpy as jnp
from jax import lax
from jax.experimental import pallas as pl
from jax.experimental.pallas import tpu as pltpu, tpu_sc as plsc

VOCAB = 1000000
B = 16384
L = 50
E = 64
PAD = 64
NC = 2
NS = 16
NW = NC * NS
BPW = B // NW
CB = 4
NCH = BPW // CB
ROT = 4
LANES = 16
NG = PAD // LANES
EUN = 4


def _sc_body(center_hbm, ctx_hbm, v_hbm, u_hbm, out_hbm,
             cidx_v, ctx_v, ubs, vbs, obs, sems, osems):
    cid = lax.axis_index("c")
    sid = lax.axis_index("s")
    wid = sid * NC + cid
    base = wid * BPW

    pltpu.sync_copy(center_hbm.at[pl.ds(base, BPW)], cidx_v)
    pltpu.sync_copy(ctx_hbm.at[pl.ds(base, BPW)], ctx_v)

    iota = lax.iota(jnp.int32, LANES)
    rowg = [iota + g * LANES for g in range(NG)]

    def issue(c, ub, sem):
        hs = []
        for i in range(CB):
            hs.append(pltpu.async_copy(u_hbm.at[ctx_v.at[c * CB + i]],
                                       ub.at[pl.ds(i * PAD, L)], sem))
        return hs

    def compute(c, ub, vb, voff, ob, osem):
        for i in range(CB):
            rows = [rowg[g] + i * PAD for g in range(NG)]
            ifull = jnp.full((LANES,), voff + i, jnp.int32)

            def ebody(t4, accs, rows=rows, ifull=ifull):
                for r in range(EUN):
                    col = (iota + (t4 * EUN + r)) & (E - 1)
                    s = plsc.load_gather(vb, [ifull, col])
                    accs = tuple(
                        accs[g] + plsc.load_gather(ub, [rows[g], col]) * s
                        for g in range(NG))
                return accs

            accs = lax.fori_loop(
                0, E // EUN, ebody,
                tuple(jnp.zeros((LANES,), jnp.float32) for _ in range(NG)))
            for g in range(NG):
                ob[i, pl.ds(g * LANES, LANES)] = accs[g]

        return pltpu.async_copy(ob, out_hbm.at[pl.ds(base + c * CB, CB)],
                                osem)

    def outer(cr, carry):
        c0 = cr * ROT
        vhs = [pltpu.async_copy(
            v_hbm.at[cidx_v.at[pl.ds((c0 + 2 * k) * CB, 2 * CB)]],
            vbs[k], sems[2 * k]) for k in range(ROT // 2)]
        hs = [issue(c0 + q, ubs[q], sems[q]) for q in range(ROT)]
        ohs = []
        for q in range(ROT):
            if q % 2 == 0:
                vhs[q // 2].wait()
            for h in hs[q]:
                h.wait()
            ohs.append(compute(c0 + q, ubs[q], vbs[q // 2], (q % 2) * CB,
                               obs[q], osems[q]))
        for oh in ohs:
            oh.wait()
        return carry

    lax.fori_loop(0, NCH // ROT, outer, 0)


def kernel(center, context_negative, v_table, u_table):
    mesh = plsc.VectorSubcoreMesh(core_axis_name="c", subcore_axis_name="s")
    padded = pl.kernel(
        _sc_body,
        out_type=jax.ShapeDtypeStruct((B, PAD), jnp.float32),
        mesh=mesh,
        compiler_params=pltpu.CompilerParams(needs_layout_passes=False,
                                             use_tc_tiling_on_sc=False),
        scratch_types=[
            pltpu.VMEM((BPW,), jnp.int32),
            pltpu.VMEM((BPW, L), jnp.int32),
            [pltpu.VMEM((CB * PAD, E), jnp.float32) for _ in range(ROT)],
            [pltpu.VMEM((2 * CB, E), jnp.float32) for _ in range(ROT // 2)],
            [pltpu.VMEM((CB, PAD), jnp.float32) for _ in range(ROT)],
            [pltpu.SemaphoreType.DMA for _ in range(ROT)],
            [pltpu.SemaphoreType.DMA for _ in range(ROT)],
        ],
    )(center.reshape(B), context_negative, v_table, u_table)
    return padded[:, :L].reshape(B, 1, L)
